# baseline (speedup 1.0000x reference)
import functools

import jax
import jax.numpy as jnp
from jax import lax
from jax.experimental import pallas as pl
from jax.experimental.pallas import tpu as pltpu
from jax.experimental.pallas import tpu_sc as plsc

N = 10000
E = 320000
H = 128
NC = 2
NS = 16
NT = NC * NS
CHUNK = 128
CPT = 80
NCHUNKS = NT * CPT
E_PAD = NCHUNKS * CHUNK
N_PAD = 10240
RPT = N_PAD // NS


def _mesh():
    return plsc.VectorSubcoreMesh(core_axis_name="c", subcore_axis_name="s")


@functools.partial(
    pl.kernel,
    mesh=_mesh(),
    out_type=jax.ShapeDtypeStruct((NC, N_PAD), jnp.float32),
    scratch_types=[
        pltpu.VMEM((CPT, CHUNK), jnp.int32),
        pltpu.VMEM((CHUNK,), jnp.float32),
        pltpu.VMEM_SHARED((N_PAD,), jnp.float32),
    ],
)
def _sc_degree(dst_hbm, zn_hbm, out_hbm, didx_all, ones, acc):
    cid = lax.axis_index("c")
    sid = lax.axis_index("s")
    wid = cid * NS + sid
    for j in range(CHUNK // 16):
        ones[pl.ds(j * 16, 16)] = jnp.ones((16,), jnp.float32)

    r0 = sid * RPT
    pltpu.sync_copy(zn_hbm.at[pl.ds(r0, RPT)], acc.at[pl.ds(r0, RPT)])
    pltpu.sync_copy(dst_hbm.at[wid], didx_all)
    plsc.subcore_barrier()

    def body(i, carry):
        pltpu.sync_copy(ones, acc.at[didx_all.at[i]], add=True)
        return carry

    lax.fori_loop(0, CPT, body, 0)

    plsc.subcore_barrier()
    pltpu.sync_copy(acc.at[pl.ds(r0, RPT)], out_hbm.at[cid].at[pl.ds(r0, RPT)])


@functools.partial(
    pl.kernel,
    mesh=_mesh(),
    out_type=jax.ShapeDtypeStruct((NC, N_PAD, H), jnp.float32),
    scratch_types=[
        pltpu.VMEM((CPT // 2, CHUNK), jnp.int32),
        pltpu.VMEM((CPT // 2, CHUNK), jnp.int32),
        pltpu.VMEM((CHUNK, H), jnp.float32),
        pltpu.VMEM((CHUNK, H), jnp.float32),
        pltpu.VMEM_SHARED((N_PAD, H), jnp.float32),
        pltpu.SemaphoreType.DMA,
        pltpu.SemaphoreType.DMA,
        pltpu.SemaphoreType.DMA,
        pltpu.SemaphoreType.DMA,
    ],
)
def _sc_scatter(y_hbm, src_hbm, dst_hbm, zm_hbm, out_hbm,
                sidx, didx, rows0, rows1, acc, sem0, sem1, ssem0, ssem1):
    cid = lax.axis_index("c")
    sid = lax.axis_index("s")
    wid = cid * NS + sid
    r0 = sid * RPT
    half = CPT // 2

    pltpu.sync_copy(zm_hbm.at[pl.ds(r0, RPT)], acc.at[pl.ds(r0, RPT)])
    plsc.subcore_barrier()

    for h in range(2):
        pltpu.sync_copy(src_hbm.at[wid].at[pl.ds(h * half, half)], sidx)
        pltpu.sync_copy(dst_hbm.at[wid].at[pl.ds(h * half, half)], didx)
        pltpu.async_copy(y_hbm.at[sidx.at[0]], rows0, sem0)
        pltpu.async_copy(y_hbm.at[sidx.at[1]], rows1, sem1)

        def body(j, carry):
            i = 2 * j
            pltpu.make_async_copy(y_hbm.at[sidx.at[i]], rows0, sem0).wait()
            pltpu.async_copy(rows0, acc.at[didx.at[i]], ssem0, add=True)
            pltpu.make_async_copy(y_hbm.at[sidx.at[i + 1]], rows1,
                                  sem1).wait()
            pltpu.async_copy(rows1, acc.at[didx.at[i + 1]], ssem1, add=True)

            @pl.when(j + 1 < half // 2)
            def _():
                pltpu.make_async_copy(rows0, acc.at[didx.at[i]],
                                      ssem0).wait()
                pltpu.async_copy(y_hbm.at[sidx.at[i + 2]], rows0, sem0)
                pltpu.make_async_copy(rows1, acc.at[didx.at[i + 1]],
                                      ssem1).wait()
                pltpu.async_copy(y_hbm.at[sidx.at[i + 3]], rows1, sem1)
            return carry

        lax.fori_loop(0, half // 2, body, 0)
        pltpu.make_async_copy(rows0, acc.at[didx.at[half - 2]],
                              ssem0).wait()
        pltpu.make_async_copy(rows1, acc.at[didx.at[half - 1]],
                              ssem1).wait()

    plsc.subcore_barrier()
    pltpu.sync_copy(acc.at[pl.ds(r0, RPT)], out_hbm.at[cid].at[pl.ds(r0, RPT)])


R = 2000
GRID = N // R


def _tc_first_body(x_ref, w_ref, da_ref, db_ref, dinv_ref, y_ref):
    deg = da_ref[0] + db_ref[0] + 1.0
    dinv = lax.rsqrt(deg)
    dinv_ref[...] = dinv
    y_ref[...] = jnp.dot(x_ref[...], w_ref[...],
                         preferred_element_type=jnp.float32) * dinv


_tc_first = pl.pallas_call(
    _tc_first_body,
    grid=(GRID,),
    in_specs=[
        pl.BlockSpec((R, H), lambda i: (i, 0)),
        pl.BlockSpec((H, H), lambda i: (0, 0)),
        pl.BlockSpec((1, R, 1), lambda i: (0, i, 0)),
        pl.BlockSpec((1, R, 1), lambda i: (1, i, 0)),
    ],
    out_specs=[
        pl.BlockSpec((R, 1), lambda i: (i, 0)),
        pl.BlockSpec((R, H), lambda i: (i, 0)),
    ],
    out_shape=[
        jax.ShapeDtypeStruct((N, 1), jnp.float32),
        jax.ShapeDtypeStruct((N, H), jnp.float32),
    ],
)


def _tc_mid_body(pa_ref, pb_ref, y_ref, dinv_ref, b_ref, w_ref, h_ref,
                 yn_ref):
    dinv = dinv_ref[...]
    h = jnp.maximum(
        dinv * (pa_ref[0] + pb_ref[0] + y_ref[...]) + b_ref[...], 0.0)
    h_ref[...] = h
    yn_ref[...] = jnp.dot(h, w_ref[...],
                          preferred_element_type=jnp.float32) * dinv


_tc_mid = pl.pallas_call(
    _tc_mid_body,
    grid=(GRID,),
    in_specs=[
        pl.BlockSpec((1, R, H), lambda i: (0, i, 0)),
        pl.BlockSpec((1, R, H), lambda i: (1, i, 0)),
        pl.BlockSpec((R, H), lambda i: (i, 0)),
        pl.BlockSpec((R, 1), lambda i: (i, 0)),
        pl.BlockSpec((1, H), lambda i: (0, 0)),
        pl.BlockSpec((H, H), lambda i: (0, 0)),
    ],
    out_specs=[
        pl.BlockSpec((R, H), lambda i: (i, 0)),
        pl.BlockSpec((R, H), lambda i: (i, 0)),
    ],
    out_shape=[
        jax.ShapeDtypeStruct((N, H), jnp.float32),
        jax.ShapeDtypeStruct((N, H), jnp.float32),
    ],
)


def _tc_last_body(pa_ref, pb_ref, y_ref, dinv_ref, b_ref, h1_ref, h2_ref,
                  jk_ref, out_ref):
    h3 = jnp.maximum(
        dinv_ref[...] * (pa_ref[0] + pb_ref[0] + y_ref[...]) + b_ref[...],
        0.0)
    e = jnp.exp(jk_ref[...])
    w = e / jnp.sum(e)
    out_ref[...] = (w[0:1, 0:1] * h1_ref[...] + w[0:1, 1:2] * h2_ref[...]
                    + w[0:1, 2:3] * h3)


_tc_last = pl.pallas_call(
    _tc_last_body,
    grid=(GRID,),
    in_specs=[
        pl.BlockSpec((1, R, H), lambda i: (0, i, 0)),
        pl.BlockSpec((1, R, H), lambda i: (1, i, 0)),
        pl.BlockSpec((R, H), lambda i: (i, 0)),
        pl.BlockSpec((R, 1), lambda i: (i, 0)),
        pl.BlockSpec((1, H), lambda i: (0, 0)),
        pl.BlockSpec((R, H), lambda i: (i, 0)),
        pl.BlockSpec((R, H), lambda i: (i, 0)),
        pl.BlockSpec((1, 3), lambda i: (0, 0)),
    ],
    out_specs=pl.BlockSpec((R, H), lambda i: (i, 0)),
    out_shape=jax.ShapeDtypeStruct((N, H), jnp.float32),
)


def kernel(x, edge_index, batch_nodes, W1, b1, W2, b2, W3, b3, jk_w):
    del batch_nodes
    ei = edge_index.astype(jnp.int32)
    pad = E_PAD - E
    iot = jnp.arange(pad, dtype=jnp.int32)
    src3d = jnp.concatenate([ei[0], iot % N]).reshape(NT, CPT, CHUNK)
    dst3d = jnp.concatenate(
        [ei[1], N + iot % (N_PAD - N)]).reshape(NT, CPT, CHUNK)
    zn = jnp.zeros((N_PAD,), jnp.float32)
    zm = jnp.zeros((N_PAD, H), jnp.float32)

    degp = _sc_degree(dst3d, zn).reshape(NC, N_PAD, 1)
    dinv, y = _tc_first(x, W1, degp, degp)

    p = _sc_scatter(y, src3d, dst3d, zm)
    h1, y = _tc_mid(p, p, y, dinv, b1.reshape(1, H), W2)
    p = _sc_scatter(y, src3d, dst3d, zm)
    h2, y = _tc_mid(p, p, y, dinv, b2.reshape(1, H), W3)
    p = _sc_scatter(y, src3d, dst3d, zm)
    out = _tc_last(p, p, y, dinv, b3.reshape(1, H), h1, h2, jk_w.reshape(1, 3))
    return out


# speedup vs baseline: 1.2718x; 1.2718x over previous
import functools

import jax
import jax.numpy as jnp
from jax import lax
from jax.experimental import pallas as pl
from jax.experimental.pallas import tpu as pltpu
from jax.experimental.pallas import tpu_sc as plsc

N = 10000
E = 320000
H = 128
NC = 2
NS = 16
NT = NC * NS
CHUNK = 128
CPT = 80
NCHUNKS = NT * CPT
E_PAD = NCHUNKS * CHUNK
N_PAD = 10240
RPT = N_PAD // NS


def _mesh():
    return plsc.VectorSubcoreMesh(core_axis_name="c", subcore_axis_name="s")


@functools.partial(
    pl.kernel,
    mesh=_mesh(),
    out_type=jax.ShapeDtypeStruct((NC, N_PAD), jnp.float32),
    scratch_types=[
        pltpu.VMEM((CPT, CHUNK), jnp.int32),
        pltpu.VMEM((CHUNK,), jnp.float32),
        pltpu.VMEM_SHARED((N_PAD,), jnp.float32),
    ],
)
def _sc_degree(dst_hbm, zn_hbm, out_hbm, didx_all, ones, acc):
    cid = lax.axis_index("c")
    sid = lax.axis_index("s")
    wid = cid * NS + sid
    for j in range(CHUNK // 16):
        ones[pl.ds(j * 16, 16)] = jnp.ones((16,), jnp.float32)

    r0 = sid * RPT
    pltpu.sync_copy(zn_hbm.at[pl.ds(r0, RPT)], acc.at[pl.ds(r0, RPT)])
    pltpu.sync_copy(dst_hbm.at[wid], didx_all)
    plsc.subcore_barrier()

    def body(i, carry):
        pltpu.sync_copy(ones, acc.at[didx_all.at[i]], add=True)
        return carry

    lax.fori_loop(0, CPT, body, 0)

    plsc.subcore_barrier()
    pltpu.sync_copy(acc.at[pl.ds(r0, RPT)], out_hbm.at[cid].at[pl.ds(r0, RPT)])


@functools.partial(
    pl.kernel,
    mesh=_mesh(),
    out_type=jax.ShapeDtypeStruct((NC, N_PAD, H), jnp.float32),
    scratch_types=[
        pltpu.VMEM((CPT // 2, CHUNK), jnp.int32),
        pltpu.VMEM((CPT // 2, CHUNK), jnp.int32),
        pltpu.VMEM((CHUNK, H), jnp.float32),
        pltpu.VMEM((CHUNK, H), jnp.float32),
        pltpu.VMEM_SHARED((N_PAD, H), jnp.float32),
        pltpu.SemaphoreType.DMA,
        pltpu.SemaphoreType.DMA,
        pltpu.SemaphoreType.DMA,
    ],
)
def _sc_scatter(y_hbm, src_hbm, dst_hbm, zm_hbm, out_hbm,
                sidx, didx, rows0, rows1, acc, sem0, sem1, isem):
    cid = lax.axis_index("c")
    sid = lax.axis_index("s")
    wid = cid * NS + sid
    r0 = sid * RPT
    half = CPT // 2

    pltpu.async_copy(zm_hbm.at[pl.ds(r0, RPT)], acc.at[pl.ds(r0, RPT)], isem)
    pltpu.sync_copy(src_hbm.at[wid].at[pl.ds(0, half)], sidx)
    pltpu.sync_copy(dst_hbm.at[wid].at[pl.ds(0, half)], didx)
    pltpu.make_async_copy(zm_hbm.at[pl.ds(r0, RPT)],
                          acc.at[pl.ds(r0, RPT)], isem).wait()
    plsc.subcore_barrier()

    for h in range(2):
        if h == 1:
            pltpu.sync_copy(src_hbm.at[wid].at[pl.ds(h * half, half)], sidx)
            pltpu.sync_copy(dst_hbm.at[wid].at[pl.ds(h * half, half)], didx)
        pltpu.async_copy(y_hbm.at[sidx.at[0]], rows0, sem0)

        def body(j, carry):
            i = 2 * j
            pltpu.async_copy(y_hbm.at[sidx.at[i + 1]], rows1, sem1)
            pltpu.make_async_copy(y_hbm.at[sidx.at[i]], rows0, sem0).wait()
            pltpu.sync_copy(rows0, acc.at[didx.at[i]], add=True)

            @pl.when(j + 1 < half // 2)
            def _():
                pltpu.async_copy(y_hbm.at[sidx.at[i + 2]], rows0, sem0)

            pltpu.make_async_copy(y_hbm.at[sidx.at[i + 1]], rows1,
                                  sem1).wait()
            pltpu.sync_copy(rows1, acc.at[didx.at[i + 1]], add=True)
            return carry

        lax.fori_loop(0, half // 2, body, 0)

    plsc.subcore_barrier()
    pltpu.sync_copy(acc.at[pl.ds(r0, RPT)], out_hbm.at[cid].at[pl.ds(r0, RPT)])


R = 2000
GRID = N // R


def _tc_first_body(x_ref, w_ref, da_ref, db_ref, dinv_ref, y_ref):
    deg = da_ref[0] + db_ref[0] + 1.0
    dinv = lax.rsqrt(deg)
    dinv_ref[...] = dinv
    y_ref[...] = jnp.dot(x_ref[...], w_ref[...],
                         preferred_element_type=jnp.float32) * dinv


_tc_first = pl.pallas_call(
    _tc_first_body,
    grid=(GRID,),
    in_specs=[
        pl.BlockSpec((R, H), lambda i: (i, 0)),
        pl.BlockSpec((H, H), lambda i: (0, 0)),
        pl.BlockSpec((1, R, 1), lambda i: (0, i, 0)),
        pl.BlockSpec((1, R, 1), lambda i: (1, i, 0)),
    ],
    out_specs=[
        pl.BlockSpec((R, 1), lambda i: (i, 0)),
        pl.BlockSpec((R, H), lambda i: (i, 0)),
    ],
    out_shape=[
        jax.ShapeDtypeStruct((N, 1), jnp.float32),
        jax.ShapeDtypeStruct((N, H), jnp.float32),
    ],
)


def _tc_mid_body(pa_ref, pb_ref, y_ref, dinv_ref, b_ref, w_ref, h_ref,
                 yn_ref):
    dinv = dinv_ref[...]
    h = jnp.maximum(
        dinv * (pa_ref[0] + pb_ref[0] + y_ref[...]) + b_ref[...], 0.0)
    h_ref[...] = h
    yn_ref[...] = jnp.dot(h, w_ref[...],
                          preferred_element_type=jnp.float32) * dinv


_tc_mid = pl.pallas_call(
    _tc_mid_body,
    grid=(GRID,),
    in_specs=[
        pl.BlockSpec((1, R, H), lambda i: (0, i, 0)),
        pl.BlockSpec((1, R, H), lambda i: (1, i, 0)),
        pl.BlockSpec((R, H), lambda i: (i, 0)),
        pl.BlockSpec((R, 1), lambda i: (i, 0)),
        pl.BlockSpec((1, H), lambda i: (0, 0)),
        pl.BlockSpec((H, H), lambda i: (0, 0)),
    ],
    out_specs=[
        pl.BlockSpec((R, H), lambda i: (i, 0)),
        pl.BlockSpec((R, H), lambda i: (i, 0)),
    ],
    out_shape=[
        jax.ShapeDtypeStruct((N, H), jnp.float32),
        jax.ShapeDtypeStruct((N, H), jnp.float32),
    ],
)


def _tc_last_body(pa_ref, pb_ref, y_ref, dinv_ref, b_ref, h1_ref, h2_ref,
                  jk_ref, out_ref):
    h3 = jnp.maximum(
        dinv_ref[...] * (pa_ref[0] + pb_ref[0] + y_ref[...]) + b_ref[...],
        0.0)
    e = jnp.exp(jk_ref[...])
    w = e / jnp.sum(e)
    out_ref[...] = (w[0:1, 0:1] * h1_ref[...] + w[0:1, 1:2] * h2_ref[...]
                    + w[0:1, 2:3] * h3)


_tc_last = pl.pallas_call(
    _tc_last_body,
    grid=(GRID,),
    in_specs=[
        pl.BlockSpec((1, R, H), lambda i: (0, i, 0)),
        pl.BlockSpec((1, R, H), lambda i: (1, i, 0)),
        pl.BlockSpec((R, H), lambda i: (i, 0)),
        pl.BlockSpec((R, 1), lambda i: (i, 0)),
        pl.BlockSpec((1, H), lambda i: (0, 0)),
        pl.BlockSpec((R, H), lambda i: (i, 0)),
        pl.BlockSpec((R, H), lambda i: (i, 0)),
        pl.BlockSpec((1, 3), lambda i: (0, 0)),
    ],
    out_specs=pl.BlockSpec((R, H), lambda i: (i, 0)),
    out_shape=jax.ShapeDtypeStruct((N, H), jnp.float32),
)


def kernel(x, edge_index, batch_nodes, W1, b1, W2, b2, W3, b3, jk_w):
    del batch_nodes
    ei = edge_index.astype(jnp.int32)
    pad = E_PAD - E
    iot = jnp.arange(pad, dtype=jnp.int32)
    src3d = jnp.concatenate([ei[0], iot % N]).reshape(NT, CPT, CHUNK)
    dst3d = jnp.concatenate(
        [ei[1], N + iot % (N_PAD - N)]).reshape(NT, CPT, CHUNK)
    zn = jnp.zeros((N_PAD,), jnp.float32)
    zm = jnp.zeros((N_PAD, H), jnp.float32)

    degp = _sc_degree(dst3d, zn).reshape(NC, N_PAD, 1)
    dinv, y = _tc_first(x, W1, degp, degp)

    p = _sc_scatter(y, src3d, dst3d, zm)
    h1, y = _tc_mid(p, p, y, dinv, b1.reshape(1, H), W2)
    p = _sc_scatter(y, src3d, dst3d, zm)
    h2, y = _tc_mid(p, p, y, dinv, b2.reshape(1, H), W3)
    p = _sc_scatter(y, src3d, dst3d, zm)
    out = _tc_last(p, p, y, dinv, b3.reshape(1, H), h1, h2, jk_w.reshape(1, 3))
    return out
